# baseline (speedup 1.0000x reference)
import functools

import jax
import jax.numpy as jnp
from jax import lax
from jax.experimental import pallas as pl
from jax.experimental.pallas import tpu as pltpu
from jax.experimental.pallas import tpu_sc as plsc

VOCAB = 100000
EMBED = 64
NFEAT = 26
HIDDEN = 1024
NCLASS = 2
BATCH = 4096

_NC = 2
_NS = 16
_NW = _NC * _NS

_ROWS = BATCH * NFEAT
_RPW = _ROWS // _NW
_CHUNK = 1664
_NCHUNK = _RPW // _CHUNK


def _gather_sc(idx, emb):
    mesh = plsc.VectorSubcoreMesh(core_axis_name="c", subcore_axis_name="s")
    rows = idx.shape[0]
    rpw = rows // _NW
    nchunk = rpw // _CHUNK

    @functools.partial(
        pl.kernel,
        mesh=mesh,
        compiler_params=pltpu.CompilerParams(use_tc_tiling_on_sc=False),
        out_type=jax.ShapeDtypeStruct((rows, EMBED), jnp.float32),
        scratch_types=[
            pltpu.VMEM((_CHUNK,), jnp.int32),
            pltpu.VMEM((_CHUNK, EMBED), jnp.float32),
            pltpu.SemaphoreType.DMA,
        ],
    )
    def k(idx_hbm, emb_hbm, out_hbm, idx_v, rows_v, sem):
        wid = lax.axis_index("s") * _NC + lax.axis_index("c")
        base = wid * rpw
        for c in range(nchunk):
            off = base + c * _CHUNK
            pltpu.sync_copy(idx_hbm.at[pl.ds(off, _CHUNK)], idx_v)
            pltpu.async_copy(emb_hbm.at[idx_v], rows_v, sem).wait()
            pltpu.sync_copy(rows_v, out_hbm.at[pl.ds(off, _CHUNK)])

    return k(idx, emb)


_BB = 1024


def _mlp_body(a_ref, w1_ref, b1_ref, w2_ref, b2_ref, o_ref):
    h = lax.dot_general(a_ref[...], w1_ref[...], (((1,), (1,)), ((), ())),
                        preferred_element_type=jnp.float32)
    h = jnp.maximum(h + b1_ref[...], 0.0)
    o = lax.dot_general(h, w2_ref[...], (((1,), (1,)), ((), ())),
                        preferred_element_type=jnp.float32)
    o_ref[...] = o + b2_ref[...]


def _mlp_tc(a, W1, b1, W2, b2):
    din = NFEAT * EMBED
    batch = a.shape[0]
    return pl.pallas_call(
        _mlp_body,
        grid=(batch // _BB,),
        in_specs=[
            pl.BlockSpec((_BB, din), lambda i: (i, 0)),
            pl.BlockSpec((HIDDEN, din), lambda i: (0, 0)),
            pl.BlockSpec((1, HIDDEN), lambda i: (0, 0)),
            pl.BlockSpec((NCLASS, HIDDEN), lambda i: (0, 0)),
            pl.BlockSpec((1, NCLASS), lambda i: (0, 0)),
        ],
        out_specs=pl.BlockSpec((_BB, NCLASS), lambda i: (i, 0)),
        out_shape=jax.ShapeDtypeStruct((batch, NCLASS), jnp.float32),
    )(a, W1, b1.reshape(1, HIDDEN), W2, b2.reshape(1, NCLASS))


def kernel(x, emb, W1, b1, W2, b2):
    flat_idx = x.reshape(-1).astype(jnp.int32)
    half = _ROWS // 2
    outs = []
    for s in range(2):
        gathered = _gather_sc(flat_idx[s * half:(s + 1) * half], emb)
        a = gathered.reshape(BATCH // 2, NFEAT * EMBED)
        outs.append(_mlp_tc(a, W1, b1, W2, b2))
    return jnp.concatenate(outs, axis=0)


# speedup vs baseline: 1.0209x; 1.0209x over previous
import functools

import jax
import jax.numpy as jnp
from jax import lax
from jax.experimental import pallas as pl
from jax.experimental.pallas import tpu as pltpu
from jax.experimental.pallas import tpu_sc as plsc

VOCAB = 100000
EMBED = 64
NFEAT = 26
HIDDEN = 1024
NCLASS = 2
BATCH = 4096

_NC = 2
_NS = 16
_NW = _NC * _NS

_ROWS = BATCH * NFEAT
_RPW = _ROWS // _NW
_CHUNK = 1664
_NCHUNK = _RPW // _CHUNK


def _gather_sc(idx, emb):
    mesh = plsc.VectorSubcoreMesh(core_axis_name="c", subcore_axis_name="s")
    rows = idx.shape[0]
    rpw = rows // _NW
    nchunk = rpw // _CHUNK

    @functools.partial(
        pl.kernel,
        mesh=mesh,
        compiler_params=pltpu.CompilerParams(use_tc_tiling_on_sc=False),
        out_type=jax.ShapeDtypeStruct((rows, EMBED), jnp.float32),
        scratch_types=[
            pltpu.VMEM((_CHUNK,), jnp.int32),
            pltpu.VMEM((_CHUNK, EMBED), jnp.float32),
            pltpu.SemaphoreType.DMA,
        ],
    )
    def k(idx_hbm, emb_hbm, out_hbm, idx_v, rows_v, sem):
        wid = lax.axis_index("s") * _NC + lax.axis_index("c")
        base = wid * rpw
        for c in range(nchunk):
            off = base + c * _CHUNK
            pltpu.sync_copy(idx_hbm.at[pl.ds(off, _CHUNK)], idx_v)
            pltpu.async_copy(emb_hbm.at[idx_v], rows_v, sem).wait()
            pltpu.sync_copy(rows_v, out_hbm.at[pl.ds(off, _CHUNK)])

    return k(idx, emb)


_BB = 1024


def _mlp_body(a_ref, w1_ref, b1_ref, w2_ref, b2_ref, o_ref):
    h = lax.dot_general(a_ref[...], w1_ref[...], (((1,), (1,)), ((), ())),
                        preferred_element_type=jnp.float32)
    h = jnp.maximum(h + b1_ref[...], 0.0)
    o = lax.dot_general(h, w2_ref[...], (((1,), (1,)), ((), ())),
                        preferred_element_type=jnp.float32)
    o_ref[...] = o + b2_ref[...]


def _mlp_tc(a, W1, b1, W2, b2):
    din = NFEAT * EMBED
    batch = a.shape[0]
    return pl.pallas_call(
        _mlp_body,
        grid=(batch // _BB,),
        in_specs=[
            pl.BlockSpec((_BB, din), lambda i: (i, 0)),
            pl.BlockSpec((HIDDEN, din), lambda i: (0, 0)),
            pl.BlockSpec((1, HIDDEN), lambda i: (0, 0)),
            pl.BlockSpec((NCLASS, HIDDEN), lambda i: (0, 0)),
            pl.BlockSpec((1, NCLASS), lambda i: (0, 0)),
        ],
        out_specs=pl.BlockSpec((_BB, NCLASS), lambda i: (i, 0)),
        out_shape=jax.ShapeDtypeStruct((batch, NCLASS), jnp.float32),
    )(a, W1, b1.reshape(1, HIDDEN), W2, b2.reshape(1, NCLASS))


def kernel(x, emb, W1, b1, W2, b2):
    flat_idx = x.reshape(-1).astype(jnp.int32)
    gathered = _gather_sc(flat_idx, emb)
    a = gathered.reshape(BATCH, NFEAT * EMBED)
    return _mlp_tc(a, W1, b1, W2, b2)
